# Initial kernel scaffold; baseline (speedup 1.0000x reference)
import functools

import jax
import jax.numpy as jnp
from jax import lax
from jax.experimental import pallas as pl
from jax.experimental.pallas import tpu as pltpu
from jax.experimental.pallas import tpu_sc as plsc

N = 10000
E = 320000
D = 128
L = 4

NC = 2
NS = 16
NW = NC * NS

IDXW = 128
ROWS_PT = 80
EPT = IDXW * ROWS_PT
E_PAD = NW * EPT
STEP = 8
NOUT = ROWS_PT // STEP

N_PAD = 10016
RPT = N_PAD // NS
DEGW = 16

BLK = 1000

_mesh = plsc.VectorSubcoreMesh(core_axis_name="c", subcore_axis_name="s")


@functools.partial(
    pl.kernel,
    out_type=jax.ShapeDtypeStruct((NC, N_PAD, D), jnp.float32),
    mesh=_mesh,
    scratch_types=[
        pltpu.VMEM((STEP, IDXW), jnp.int32),
        pltpu.VMEM((STEP, IDXW), jnp.int32),
        pltpu.VMEM((IDXW, D), jnp.float32),
        pltpu.VMEM_SHARED((N_PAD, D), jnp.float32),
    ],
)
def _sc_segsum(h_hbm, src_hbm, dst_hbm, z_hbm, out_hbm, sidx, didx, rows, acc):
    c = lax.axis_index("c")
    s = lax.axis_index("s")
    pltpu.sync_copy(z_hbm, acc.at[pl.ds(s * RPT, RPT)])
    plsc.subcore_barrier()

    base = (c * NS + s) * ROWS_PT

    @pl.loop(0, NOUT)
    def _(t):
        r0 = base + t * STEP
        pltpu.sync_copy(src_hbm.at[pl.ds(r0, STEP)], sidx)
        pltpu.sync_copy(dst_hbm.at[pl.ds(r0, STEP)], didx)
        for j in range(STEP):
            pltpu.sync_copy(h_hbm.at[sidx.at[j]], rows)
            pltpu.sync_copy(rows, acc.at[didx.at[j]], add=True)

    plsc.subcore_barrier()
    pltpu.sync_copy(acc.at[pl.ds(s * RPT, RPT)],
                    out_hbm.at[c, pl.ds(s * RPT, RPT)])


@functools.partial(
    pl.kernel,
    out_type=jax.ShapeDtypeStruct((NC, N_PAD, DEGW), jnp.float32),
    mesh=_mesh,
    scratch_types=[
        pltpu.VMEM((STEP, IDXW), jnp.int32),
        pltpu.VMEM((IDXW, DEGW), jnp.float32),
        pltpu.VMEM_SHARED((N_PAD, DEGW), jnp.float32),
    ],
)
def _sc_degree(dst_hbm, ones_hbm, z_hbm, out_hbm, didx, ones_v, acc):
    c = lax.axis_index("c")
    s = lax.axis_index("s")
    pltpu.sync_copy(z_hbm, acc.at[pl.ds(s * RPT, RPT)])
    pltpu.sync_copy(ones_hbm, ones_v)
    plsc.subcore_barrier()

    base = (c * NS + s) * ROWS_PT

    @pl.loop(0, NOUT)
    def _(t):
        r0 = base + t * STEP
        pltpu.sync_copy(dst_hbm.at[pl.ds(r0, STEP)], didx)
        for j in range(STEP):
            pltpu.sync_copy(ones_v, acc.at[didx.at[j]], add=True)

    plsc.subcore_barrier()
    pltpu.sync_copy(acc.at[pl.ds(s * RPT, RPT)],
                    out_hbm.at[c, pl.ds(s * RPT, RPT)])


def _tc_proj(x, W, b):
    def body(x_ref, w_ref, b_ref, o_ref):
        o_ref[...] = (
            jnp.dot(x_ref[...], w_ref[...], preferred_element_type=jnp.float32)
            + b_ref[...]
        )

    return pl.pallas_call(
        body,
        grid=(N // BLK,),
        in_specs=[
            pl.BlockSpec((BLK, D), lambda i: (i, 0)),
            pl.BlockSpec((D, D), lambda i: (0, 0)),
            pl.BlockSpec((1, D), lambda i: (0, 0)),
        ],
        out_specs=pl.BlockSpec((BLK, D), lambda i: (i, 0)),
        out_shape=jax.ShapeDtypeStruct((N, D), jnp.float32),
    )(x, W, b)


def _tc_layer(parts, deg_parts, h, Wl_i, Wr_i, bl_i, g_i, beta_i):
    def body(p_ref, dp_ref, h_ref, wl_ref, wr_ref, bl_ref, g_ref, be_ref,
             o_ref):
        deg = jnp.maximum(dp_ref[0, :, 0:1] + dp_ref[1, :, 0:1], 1.0)
        msg = (p_ref[0] + p_ref[1]) / deg
        out = (
            jnp.dot(msg, wl_ref[...], preferred_element_type=jnp.float32)
            + bl_ref[...]
            + jnp.dot(h_ref[...], wr_ref[...],
                      preferred_element_type=jnp.float32)
        )
        out = out * jax.nn.sigmoid(out)
        mu = jnp.mean(out, axis=1, keepdims=True)
        var = jnp.mean((out - mu) ** 2, axis=1, keepdims=True)
        o_ref[...] = (out - mu) * lax.rsqrt(var + 1e-5) * g_ref[...] \
            + be_ref[...]

    return pl.pallas_call(
        body,
        grid=(N // BLK,),
        in_specs=[
            pl.BlockSpec((NC, BLK, D), lambda i: (0, i, 0)),
            pl.BlockSpec((NC, BLK, DEGW), lambda i: (0, i, 0)),
            pl.BlockSpec((BLK, D), lambda i: (i, 0)),
            pl.BlockSpec((D, D), lambda i: (0, 0)),
            pl.BlockSpec((D, D), lambda i: (0, 0)),
            pl.BlockSpec((1, D), lambda i: (0, 0)),
            pl.BlockSpec((1, D), lambda i: (0, 0)),
            pl.BlockSpec((1, D), lambda i: (0, 0)),
        ],
        out_specs=pl.BlockSpec((BLK, D), lambda i: (i, 0)),
        out_shape=jax.ShapeDtypeStruct((N, D), jnp.float32),
    )(parts, deg_parts, h, Wl_i, Wr_i, bl_i, g_i, beta_i)


def kernel(x, edge_index, W_in, b_in, Wl, bl, Wr, g, beta):
    src = edge_index[0]
    dst = edge_index[1]
    npad = E_PAD - E
    src2 = jnp.concatenate([src, jnp.zeros((npad,), jnp.int32)])
    src2 = src2.reshape(E_PAD // IDXW, IDXW)
    dst2 = jnp.concatenate([dst, jnp.full((npad,), N, jnp.int32)])
    dst2 = dst2.reshape(E_PAD // IDXW, IDXW)

    zeros_msg = jnp.zeros((RPT, D), jnp.float32)
    zeros_deg = jnp.zeros((RPT, DEGW), jnp.float32)
    ones_deg = jnp.ones((IDXW, DEGW), jnp.float32)

    deg_parts = _sc_degree(dst2, ones_deg, zeros_deg)
    h = _tc_proj(x, W_in, b_in.reshape(1, D))
    for i in range(L):
        parts = _sc_segsum(h, src2, dst2, zeros_msg)
        h = _tc_layer(parts, deg_parts, h, Wl[i], Wr[i],
                      bl[i].reshape(1, D), g[i].reshape(1, D),
                      beta[i].reshape(1, D))
    return h


# speedup vs baseline: 2.8179x; 2.8179x over previous
import functools

import jax
import jax.numpy as jnp
from jax import lax
from jax.experimental import pallas as pl
from jax.experimental.pallas import tpu as pltpu
from jax.experimental.pallas import tpu_sc as plsc

N = 10000
E = 320000
D = 128
L = 4

NC = 2
NS = 16
NW = NC * NS

IDXW = 128
ROWS_PT = 80
EPT = IDXW * ROWS_PT
E_PAD = NW * EPT
STEP = 8
NOUT = ROWS_PT // STEP

N_PAD = 10112
RPT = N_PAD // NS
DEGW = 128

BLK = 1000

_mesh = plsc.VectorSubcoreMesh(core_axis_name="c", subcore_axis_name="s")


@functools.partial(
    pl.kernel,
    out_type=jax.ShapeDtypeStruct((NC, N_PAD, D), jnp.float32),
    mesh=_mesh,
    scratch_types=[
        pltpu.VMEM((STEP, IDXW), jnp.int32),
        pltpu.VMEM((STEP, IDXW), jnp.int32),
        pltpu.VMEM((IDXW, D), jnp.float32),
        pltpu.VMEM_SHARED((N_PAD, D), jnp.float32),
    ],
)
def _sc_segsum(h_hbm, src_hbm, dst_hbm, z_hbm, out_hbm, sidx, didx, rows, acc):
    c = lax.axis_index("c")
    s = lax.axis_index("s")
    pltpu.sync_copy(z_hbm, acc.at[pl.ds(s * RPT, RPT)])
    plsc.subcore_barrier()

    base = (c * NS + s) * ROWS_PT

    @pl.loop(0, NOUT)
    def _(t):
        r0 = base + t * STEP
        pltpu.sync_copy(src_hbm.at[pl.ds(r0, STEP)], sidx)
        pltpu.sync_copy(dst_hbm.at[pl.ds(r0, STEP)], didx)
        for j in range(STEP):
            pltpu.sync_copy(h_hbm.at[sidx.at[j]], rows)
            pltpu.sync_copy(rows, acc.at[didx.at[j]], add=True)

    plsc.subcore_barrier()
    pltpu.sync_copy(acc.at[pl.ds(s * RPT, RPT)],
                    out_hbm.at[c, pl.ds(s * RPT, RPT)])


@functools.partial(
    pl.kernel,
    out_type=jax.ShapeDtypeStruct((NC, N_PAD, DEGW), jnp.float32),
    mesh=_mesh,
    scratch_types=[
        pltpu.VMEM((STEP, IDXW), jnp.int32),
        pltpu.VMEM((IDXW, DEGW), jnp.float32),
        pltpu.VMEM_SHARED((N_PAD, DEGW), jnp.float32),
    ],
)
def _sc_degree(dst_hbm, ones_hbm, z_hbm, out_hbm, didx, ones_v, acc):
    c = lax.axis_index("c")
    s = lax.axis_index("s")
    pltpu.sync_copy(z_hbm, acc.at[pl.ds(s * RPT, RPT)])
    pltpu.sync_copy(ones_hbm, ones_v)
    plsc.subcore_barrier()

    base = (c * NS + s) * ROWS_PT

    @pl.loop(0, NOUT)
    def _(t):
        r0 = base + t * STEP
        pltpu.sync_copy(dst_hbm.at[pl.ds(r0, STEP)], didx)
        for j in range(STEP):
            pltpu.sync_copy(ones_v, acc.at[didx.at[j]], add=True)

    plsc.subcore_barrier()
    pltpu.sync_copy(acc.at[pl.ds(s * RPT, RPT)],
                    out_hbm.at[c, pl.ds(s * RPT, RPT)])


def _tc_proj(x, W, b):
    def body(x_ref, w_ref, b_ref, o_ref):
        o_ref[...] = (
            jnp.dot(x_ref[...], w_ref[...], preferred_element_type=jnp.float32)
            + b_ref[...]
        )

    return pl.pallas_call(
        body,
        grid=(N // BLK,),
        in_specs=[
            pl.BlockSpec((BLK, D), lambda i: (i, 0)),
            pl.BlockSpec((D, D), lambda i: (0, 0)),
            pl.BlockSpec((1, D), lambda i: (0, 0)),
        ],
        out_specs=pl.BlockSpec((BLK, D), lambda i: (i, 0)),
        out_shape=jax.ShapeDtypeStruct((N, D), jnp.float32),
    )(x, W, b)


def _tc_layer(parts, deg_parts, h, Wl_i, Wr_i, bl_i, g_i, beta_i):
    def body(p_ref, dp_ref, h_ref, wl_ref, wr_ref, bl_ref, g_ref, be_ref,
             o_ref):
        deg = jnp.maximum(dp_ref[0, :, 0:1] + dp_ref[1, :, 0:1], 1.0)
        msg = (p_ref[0] + p_ref[1]) / deg
        out = (
            jnp.dot(msg, wl_ref[...], preferred_element_type=jnp.float32)
            + bl_ref[...]
            + jnp.dot(h_ref[...], wr_ref[...],
                      preferred_element_type=jnp.float32)
        )
        out = out * jax.nn.sigmoid(out)
        mu = jnp.mean(out, axis=1, keepdims=True)
        var = jnp.mean((out - mu) ** 2, axis=1, keepdims=True)
        o_ref[...] = (out - mu) * lax.rsqrt(var + 1e-5) * g_ref[...] \
            + be_ref[...]

    return pl.pallas_call(
        body,
        grid=(N // BLK,),
        in_specs=[
            pl.BlockSpec((NC, BLK, D), lambda i: (0, i, 0)),
            pl.BlockSpec((NC, BLK, DEGW), lambda i: (0, i, 0)),
            pl.BlockSpec((BLK, D), lambda i: (i, 0)),
            pl.BlockSpec((D, D), lambda i: (0, 0)),
            pl.BlockSpec((D, D), lambda i: (0, 0)),
            pl.BlockSpec((1, D), lambda i: (0, 0)),
            pl.BlockSpec((1, D), lambda i: (0, 0)),
            pl.BlockSpec((1, D), lambda i: (0, 0)),
        ],
        out_specs=pl.BlockSpec((BLK, D), lambda i: (i, 0)),
        out_shape=jax.ShapeDtypeStruct((N, D), jnp.float32),
    )(parts, deg_parts, h, Wl_i, Wr_i, bl_i, g_i, beta_i)


def kernel(x, edge_index, W_in, b_in, Wl, bl, Wr, g, beta):
    src = edge_index[0]
    dst = edge_index[1]
    npad = E_PAD - E
    src2 = jnp.concatenate([src, jnp.zeros((npad,), jnp.int32)])
    src2 = src2.reshape(E_PAD // IDXW, IDXW)
    dst2 = jnp.concatenate([dst, jnp.full((npad,), N, jnp.int32)])
    dst2 = dst2.reshape(E_PAD // IDXW, IDXW)

    zeros_msg = jnp.zeros((RPT, D), jnp.float32)
    zeros_deg = jnp.zeros((RPT, DEGW), jnp.float32)
    ones_deg = jnp.ones((IDXW, DEGW), jnp.float32)

    deg_parts = _sc_degree(dst2, ones_deg, zeros_deg)
    h = _tc_proj(x, W_in, b_in.reshape(1, D))
    for i in range(L):
        parts = _sc_segsum(h, src2, dst2, zeros_msg)
        h = _tc_layer(parts, deg_parts, h, Wl[i], Wr[i],
                      bl[i].reshape(1, D), g[i].reshape(1, D),
                      beta[i].reshape(1, D))
    return h


# speedup vs baseline: 7.4716x; 2.6515x over previous
import functools

import jax
import jax.numpy as jnp
from jax import lax
from jax.experimental import pallas as pl
from jax.experimental.pallas import tpu as pltpu
from jax.experimental.pallas import tpu_sc as plsc

N = 10000
E = 320000
D = 128
L = 4

NC = 2
NS = 16
NW = NC * NS

IDXW = 128
ROWS_PT = 80
EPT = IDXW * ROWS_PT
E_PAD = NW * EPT
STEP = 8
NOUT = ROWS_PT // STEP

N_PAD = 10112
RPT = N_PAD // NS
DEGW = 128

BLK = 1000

_mesh = plsc.VectorSubcoreMesh(core_axis_name="c", subcore_axis_name="s")


@functools.partial(
    pl.kernel,
    out_type=jax.ShapeDtypeStruct((NC, N_PAD, D), jnp.float32),
    mesh=_mesh,
    scratch_types=[
        pltpu.VMEM((STEP, IDXW), jnp.int32),
        pltpu.VMEM((STEP, IDXW), jnp.int32),
        pltpu.VMEM((IDXW, D), jnp.float32),
        pltpu.VMEM_SHARED((N_PAD, D), jnp.float32),
    ],
)
def _sc_segsum(h_hbm, src_hbm, dst_hbm, z_hbm, out_hbm, sidx, didx, rows, acc):
    c = lax.axis_index("c")
    s = lax.axis_index("s")
    pltpu.sync_copy(z_hbm, acc.at[pl.ds(s * RPT, RPT)])
    plsc.subcore_barrier()

    base = (c * NS + s) * ROWS_PT

    @pl.loop(0, NOUT)
    def _(t):
        r0 = base + t * STEP
        pltpu.sync_copy(src_hbm.at[pl.ds(r0, STEP)], sidx)
        pltpu.sync_copy(dst_hbm.at[pl.ds(r0, STEP)], didx)
        for j in range(STEP):
            pltpu.sync_copy(h_hbm.at[sidx.at[j]], rows)
            pltpu.sync_copy(rows, acc.at[didx.at[j]], add=True)

    plsc.subcore_barrier()
    pltpu.sync_copy(acc.at[pl.ds(s * RPT, RPT)],
                    out_hbm.at[c, pl.ds(s * RPT, RPT)])


@functools.partial(
    pl.kernel,
    out_type=jax.ShapeDtypeStruct((NC, N_PAD, DEGW), jnp.float32),
    mesh=_mesh,
    scratch_types=[
        pltpu.VMEM((STEP, IDXW), jnp.int32),
        pltpu.VMEM((IDXW, DEGW), jnp.float32),
        pltpu.VMEM_SHARED((N_PAD, DEGW), jnp.float32),
    ],
)
def _sc_degree(dst_hbm, ones_hbm, z_hbm, out_hbm, didx, ones_v, acc):
    c = lax.axis_index("c")
    s = lax.axis_index("s")
    pltpu.sync_copy(z_hbm, acc.at[pl.ds(s * RPT, RPT)])
    pltpu.sync_copy(ones_hbm, ones_v)
    plsc.subcore_barrier()

    base = (c * NS + s) * ROWS_PT

    @pl.loop(0, NOUT)
    def _(t):
        r0 = base + t * STEP
        pltpu.sync_copy(dst_hbm.at[pl.ds(r0, STEP)], didx)
        for j in range(STEP):
            pltpu.sync_copy(ones_v, acc.at[didx.at[j]], add=True)

    plsc.subcore_barrier()
    pltpu.sync_copy(acc.at[pl.ds(s * RPT, RPT)],
                    out_hbm.at[c, pl.ds(s * RPT, RPT)])


def _tc_proj(x, W, b):
    def body(x_ref, w_ref, b_ref, o_ref):
        o_ref[...] = (
            jnp.dot(x_ref[...], w_ref[...], preferred_element_type=jnp.float32)
            + b_ref[...]
        )

    return pl.pallas_call(
        body,
        grid=(N // BLK,),
        in_specs=[
            pl.BlockSpec((BLK, D), lambda i: (i, 0)),
            pl.BlockSpec((D, D), lambda i: (0, 0)),
            pl.BlockSpec((1, D), lambda i: (0, 0)),
        ],
        out_specs=pl.BlockSpec((BLK, D), lambda i: (i, 0)),
        out_shape=jax.ShapeDtypeStruct((N, D), jnp.float32),
    )(x, W, b)


def _tc_layer(parts, deg_parts, h, Wl_i, Wr_i, bl_i, g_i, beta_i):
    def body(p_ref, dp_ref, h_ref, wl_ref, wr_ref, bl_ref, g_ref, be_ref,
             o_ref):
        deg = jnp.maximum(dp_ref[0, :, 0:1] + dp_ref[1, :, 0:1], 1.0)
        msg = (p_ref[0] + p_ref[1]) / deg
        out = (
            jnp.dot(msg, wl_ref[...], preferred_element_type=jnp.float32)
            + bl_ref[...]
            + jnp.dot(h_ref[...], wr_ref[...],
                      preferred_element_type=jnp.float32)
        )
        out = out * jax.nn.sigmoid(out)
        mu = jnp.mean(out, axis=1, keepdims=True)
        var = jnp.mean((out - mu) ** 2, axis=1, keepdims=True)
        o_ref[...] = (out - mu) * lax.rsqrt(var + 1e-5) * g_ref[...] \
            + be_ref[...]

    return pl.pallas_call(
        body,
        grid=(N // BLK,),
        in_specs=[
            pl.BlockSpec((NC, BLK, D), lambda i: (0, i, 0)),
            pl.BlockSpec((NC, BLK, DEGW), lambda i: (0, i, 0)),
            pl.BlockSpec((BLK, D), lambda i: (i, 0)),
            pl.BlockSpec((D, D), lambda i: (0, 0)),
            pl.BlockSpec((D, D), lambda i: (0, 0)),
            pl.BlockSpec((1, D), lambda i: (0, 0)),
            pl.BlockSpec((1, D), lambda i: (0, 0)),
            pl.BlockSpec((1, D), lambda i: (0, 0)),
        ],
        out_specs=pl.BlockSpec((BLK, D), lambda i: (i, 0)),
        out_shape=jax.ShapeDtypeStruct((N, D), jnp.float32),
    )(parts, deg_parts, h, Wl_i, Wr_i, bl_i, g_i, beta_i)


def kernel(x, edge_index, W_in, b_in, Wl, bl, Wr, g, beta):
    src = edge_index[0]
    dst = edge_index[1]
    npad = E_PAD - E
    pad_src = jnp.arange(npad, dtype=jnp.int32) % N
    pad_dst = N + jnp.arange(npad, dtype=jnp.int32) % (N_PAD - N)
    src2 = jnp.concatenate([src, pad_src]).reshape(E_PAD // IDXW, IDXW)
    dst2 = jnp.concatenate([dst, pad_dst]).reshape(E_PAD // IDXW, IDXW)

    zeros_msg = jnp.zeros((RPT, D), jnp.float32)
    zeros_deg = jnp.zeros((RPT, DEGW), jnp.float32)
    ones_deg = jnp.ones((IDXW, DEGW), jnp.float32)

    deg_parts = _sc_degree(dst2, ones_deg, zeros_deg)
    h = _tc_proj(x, W_in, b_in.reshape(1, D))
    for i in range(L):
        parts = _sc_segsum(h, src2, dst2, zeros_msg)
        h = _tc_layer(parts, deg_parts, h, Wl[i], Wr[i],
                      bl[i].reshape(1, D), g[i].reshape(1, D),
                      beta[i].reshape(1, D))
    return h


# speedup vs baseline: 9.7789x; 1.3088x over previous
import functools

import jax
import jax.numpy as jnp
from jax import lax
from jax.experimental import pallas as pl
from jax.experimental.pallas import tpu as pltpu
from jax.experimental.pallas import tpu_sc as plsc

N = 10000
E = 320000
D = 128
L = 4

NC = 2
NS = 16
NW = NC * NS

IDXW = 128
ROWS_PT = 80
EPT = IDXW * ROWS_PT
E_PAD = NW * EPT
STEP = 8
NOUT = ROWS_PT // STEP

N_PAD = 10112
RPT = N_PAD // NS
DEGW = 128

BLK = 1000

_mesh = plsc.VectorSubcoreMesh(core_axis_name="c", subcore_axis_name="s")


@functools.partial(
    pl.kernel,
    out_type=jax.ShapeDtypeStruct((NC, N_PAD, D), jnp.float32),
    mesh=_mesh,
    scratch_types=[
        pltpu.VMEM((ROWS_PT // 2, IDXW), jnp.int32),
        pltpu.VMEM((ROWS_PT // 2, IDXW), jnp.int32),
        pltpu.VMEM((IDXW, D), jnp.float32),
        pltpu.VMEM((IDXW, D), jnp.float32),
        pltpu.VMEM_SHARED((N_PAD, D), jnp.float32),
        pltpu.SemaphoreType.DMA,
        pltpu.SemaphoreType.DMA,
        pltpu.SemaphoreType.DMA,
        pltpu.SemaphoreType.DMA,
    ],
)
def _sc_segsum(h_hbm, src_hbm, dst_hbm, z_hbm, out_hbm,
               sidx, didx, rows0, rows1, acc, gs0, gs1, ss0, ss1):
    c = lax.axis_index("c")
    s = lax.axis_index("s")
    base = (c * NS + s) * ROWS_PT
    pltpu.sync_copy(z_hbm, acc.at[pl.ds(s * RPT, RPT)])
    plsc.subcore_barrier()

    def gather(t, buf, sem):
        pltpu.async_copy(h_hbm.at[sidx.at[t]], buf, sem)

    def gather_wait(t, buf, sem):
        pltpu.make_async_copy(h_hbm.at[sidx.at[t]], buf, sem).wait()

    def scat(t, buf, sem):
        pltpu.async_copy(buf, acc.at[didx.at[t]], sem, add=True)

    def scat_wait(t, buf, sem):
        pltpu.make_async_copy(buf, acc.at[didx.at[t]], sem).wait()

    HALF = ROWS_PT // 2
    for hf in range(2):
        pltpu.sync_copy(src_hbm.at[pl.ds(base + hf * HALF, HALF)], sidx)
        pltpu.sync_copy(dst_hbm.at[pl.ds(base + hf * HALF, HALF)], didx)

        gather(0, rows0, gs0)
        gather_wait(0, rows0, gs0)
        scat(0, rows0, ss0)
        gather(1, rows1, gs1)

        @pl.loop(1, HALF - 1, step=2)
        def _(t):
            gather_wait(t, rows1, gs1)
            scat_wait(t - 1, rows0, ss0)
            gather(t + 1, rows0, gs0)
            scat(t, rows1, ss1)
            gather_wait(t + 1, rows0, gs0)
            scat_wait(t, rows1, ss1)
            gather(t + 2, rows1, gs1)
            scat(t + 1, rows0, ss0)

        gather_wait(HALF - 1, rows1, gs1)
        scat_wait(HALF - 2, rows0, ss0)
        scat(HALF - 1, rows1, ss1)
        scat_wait(HALF - 1, rows1, ss1)

    plsc.subcore_barrier()
    pltpu.sync_copy(acc.at[pl.ds(s * RPT, RPT)],
                    out_hbm.at[c, pl.ds(s * RPT, RPT)])


@functools.partial(
    pl.kernel,
    out_type=jax.ShapeDtypeStruct((NC, N_PAD, DEGW), jnp.float32),
    mesh=_mesh,
    scratch_types=[
        pltpu.VMEM((STEP, IDXW), jnp.int32),
        pltpu.VMEM((IDXW, DEGW), jnp.float32),
        pltpu.VMEM_SHARED((N_PAD, DEGW), jnp.float32),
    ],
)
def _sc_degree(dst_hbm, ones_hbm, z_hbm, out_hbm, didx, ones_v, acc):
    c = lax.axis_index("c")
    s = lax.axis_index("s")
    pltpu.sync_copy(z_hbm, acc.at[pl.ds(s * RPT, RPT)])
    pltpu.sync_copy(ones_hbm, ones_v)
    plsc.subcore_barrier()

    base = (c * NS + s) * ROWS_PT

    @pl.loop(0, NOUT)
    def _(t):
        r0 = base + t * STEP
        pltpu.sync_copy(dst_hbm.at[pl.ds(r0, STEP)], didx)
        for j in range(STEP):
            pltpu.sync_copy(ones_v, acc.at[didx.at[j]], add=True)

    plsc.subcore_barrier()
    pltpu.sync_copy(acc.at[pl.ds(s * RPT, RPT)],
                    out_hbm.at[c, pl.ds(s * RPT, RPT)])


def _tc_proj(x, W, b):
    def body(x_ref, w_ref, b_ref, o_ref):
        o_ref[...] = (
            jnp.dot(x_ref[...], w_ref[...], preferred_element_type=jnp.float32)
            + b_ref[...]
        )

    return pl.pallas_call(
        body,
        grid=(N // BLK,),
        in_specs=[
            pl.BlockSpec((BLK, D), lambda i: (i, 0)),
            pl.BlockSpec((D, D), lambda i: (0, 0)),
            pl.BlockSpec((1, D), lambda i: (0, 0)),
        ],
        out_specs=pl.BlockSpec((BLK, D), lambda i: (i, 0)),
        out_shape=jax.ShapeDtypeStruct((N, D), jnp.float32),
    )(x, W, b)


def _tc_layer(parts, deg_parts, h, Wl_i, Wr_i, bl_i, g_i, beta_i):
    def body(p_ref, dp_ref, h_ref, wl_ref, wr_ref, bl_ref, g_ref, be_ref,
             o_ref):
        deg = jnp.maximum(dp_ref[0, :, 0:1] + dp_ref[1, :, 0:1], 1.0)
        msg = (p_ref[0] + p_ref[1]) / deg
        out = (
            jnp.dot(msg, wl_ref[...], preferred_element_type=jnp.float32)
            + bl_ref[...]
            + jnp.dot(h_ref[...], wr_ref[...],
                      preferred_element_type=jnp.float32)
        )
        out = out * jax.nn.sigmoid(out)
        mu = jnp.mean(out, axis=1, keepdims=True)
        var = jnp.mean((out - mu) ** 2, axis=1, keepdims=True)
        o_ref[...] = (out - mu) * lax.rsqrt(var + 1e-5) * g_ref[...] \
            + be_ref[...]

    return pl.pallas_call(
        body,
        grid=(N // BLK,),
        in_specs=[
            pl.BlockSpec((NC, BLK, D), lambda i: (0, i, 0)),
            pl.BlockSpec((NC, BLK, DEGW), lambda i: (0, i, 0)),
            pl.BlockSpec((BLK, D), lambda i: (i, 0)),
            pl.BlockSpec((D, D), lambda i: (0, 0)),
            pl.BlockSpec((D, D), lambda i: (0, 0)),
            pl.BlockSpec((1, D), lambda i: (0, 0)),
            pl.BlockSpec((1, D), lambda i: (0, 0)),
            pl.BlockSpec((1, D), lambda i: (0, 0)),
        ],
        out_specs=pl.BlockSpec((BLK, D), lambda i: (i, 0)),
        out_shape=jax.ShapeDtypeStruct((N, D), jnp.float32),
    )(parts, deg_parts, h, Wl_i, Wr_i, bl_i, g_i, beta_i)


def kernel(x, edge_index, W_in, b_in, Wl, bl, Wr, g, beta):
    src = edge_index[0]
    dst = edge_index[1]
    npad = E_PAD - E
    pad_src = jnp.arange(npad, dtype=jnp.int32) % N
    pad_dst = N + jnp.arange(npad, dtype=jnp.int32) % (N_PAD - N)
    src2 = jnp.concatenate([src, pad_src]).reshape(E_PAD // IDXW, IDXW)
    dst2 = jnp.concatenate([dst, pad_dst]).reshape(E_PAD // IDXW, IDXW)

    zeros_msg = jnp.zeros((RPT, D), jnp.float32)
    zeros_deg = jnp.zeros((RPT, DEGW), jnp.float32)
    ones_deg = jnp.ones((IDXW, DEGW), jnp.float32)

    deg_parts = _sc_degree(dst2, ones_deg, zeros_deg)
    h = _tc_proj(x, W_in, b_in.reshape(1, D))
    for i in range(L):
        parts = _sc_segsum(h, src2, dst2, zeros_msg)
        h = _tc_layer(parts, deg_parts, h, Wl[i], Wr[i],
                      bl[i].reshape(1, D), g[i].reshape(1, D),
                      beta[i].reshape(1, D))
    return h
